# baseline (speedup 1.0000x reference)
import functools

import jax
import jax.numpy as jnp
from jax import lax
from jax.experimental import pallas as pl
from jax.experimental.pallas import tpu as pltpu
from jax.experimental.pallas import tpu_sc as plsc

NC = 2
NS = 16
NW = NC * NS
CHUNK = 128
NB = 4
RB = 2000


def _sc_mesh():
    return plsc.VectorSubcoreMesh(core_axis_name="c", subcore_axis_name="s")


def _make_spmm(n_acc, nch, width, nb=2):
    rpt = n_acc // NS
    if nb == 1:
        nwin = 1
    elif nb == 3 and nch % 3 == 0:
        nwin = 3
    else:
        nwin = 2 if nch % 2 == 0 else 1
    win = nch // nwin
    assert win % nb == 0

    @functools.partial(
        pl.kernel,
        mesh=_sc_mesh(),
        out_type=jax.ShapeDtypeStruct((NC, n_acc, width), jnp.float32),
        scratch_types=(
            [pltpu.VMEM((win, CHUNK), jnp.int32),
             pltpu.VMEM((win, CHUNK), jnp.int32)]
            + [pltpu.VMEM((CHUNK, width), jnp.float32)] * nb
            + [pltpu.VMEM_SHARED((n_acc, width), jnp.float32)]
            + [pltpu.SemaphoreType.DMA] * (2 * nb)
        ),
    )
    def spmm(table, src3, dst3, zeros, out, srcv, dstv, *rest):
        rows = rest[:nb]
        acc = rest[nb]
        gsem = rest[nb + 1:nb + 1 + nb]
        ssem = rest[nb + 1 + nb:]
        cid = lax.axis_index("c")
        sid = lax.axis_index("s")
        wid = sid * NC + cid
        sl = pl.ds(sid * rpt, rpt)
        pltpu.sync_copy(zeros.at[sl], acc.at[sl])
        plsc.subcore_barrier()

        for w in range(nwin):
            if nwin == 1:
                pltpu.sync_copy(src3.at[wid], srcv)
                pltpu.sync_copy(dst3.at[wid], dstv)
            else:
                pltpu.sync_copy(src3.at[wid, pl.ds(w * win, win)], srcv)
                pltpu.sync_copy(dst3.at[wid, pl.ds(w * win, win)], dstv)

            if nb == 1:
                def body(j, carry):
                    pltpu.async_copy(
                        table.at[srcv.at[j]], rows[0], gsem[0]).wait()
                    pltpu.sync_copy(rows[0], acc.at[dstv.at[j]], add=True)
                    return carry

                lax.fori_loop(0, win, body, 0)
            else:
                for b in range(nb):
                    pltpu.async_copy(table.at[srcv.at[b]], rows[b], gsem[b])

                def group(gi, carry):
                    for b in range(nb):
                        j = gi * nb + b
                        pltpu.make_async_copy(
                            table.at[srcv.at[j]], rows[b], gsem[b]).wait()
                        pltpu.async_copy(
                            rows[b], acc.at[dstv.at[j]], ssem[b], add=True)
                    for b in range(nb):
                        j = gi * nb + b
                        pltpu.make_async_copy(
                            rows[b], acc.at[dstv.at[j]], ssem[b]).wait()
                        jn = jnp.minimum(j + nb, win - 1)
                        pltpu.async_copy(
                            table.at[srcv.at[jn]], rows[b], gsem[b])
                    return carry

                lax.fori_loop(0, win // nb, group, 0)
                for b in range(nb):
                    pltpu.make_async_copy(
                        table.at[srcv.at[0]], rows[b], gsem[b]).wait()

        plsc.subcore_barrier()
        pltpu.sync_copy(acc.at[sl], out.at[cid, sl])

    return spmm


def _make_deg(n_acc, nch):
    rpt = n_acc // NS

    @functools.partial(
        pl.kernel,
        mesh=_sc_mesh(),
        out_type=jax.ShapeDtypeStruct((NC, n_acc, 128), jnp.float32),
        scratch_types=[
            pltpu.VMEM((nch, CHUNK), jnp.int32),
            pltpu.VMEM((CHUNK, 128), jnp.float32),
            pltpu.VMEM_SHARED((n_acc, 128), jnp.float32),
            pltpu.SemaphoreType.DMA,
        ],
    )
    def deg(dst3, ones, zeros, out, dstv, onesv, acc, sem):
        cid = lax.axis_index("c")
        sid = lax.axis_index("s")
        wid = sid * NC + cid
        sl = pl.ds(sid * rpt, rpt)
        pltpu.sync_copy(zeros.at[sl], acc.at[sl])
        pltpu.sync_copy(dst3.at[wid], dstv)
        pltpu.sync_copy(ones, onesv)
        plsc.subcore_barrier()

        k = next(d for d in (10, 9, 8, 7, 6, 5, 4, 3, 2, 1) if nch % d == 0)
        def group(gi, carry):
            for b in range(k):
                pltpu.async_copy(
                    onesv, acc.at[dstv.at[gi * k + b]], sem, add=True)
            for b in range(k):
                pltpu.make_async_copy(
                    onesv, acc.at[dstv.at[0]], sem).wait()
            return carry

        lax.fori_loop(0, nch // k, group, 0)
        plsc.subcore_barrier()
        pltpu.sync_copy(acc.at[sl], out.at[cid, sl])

    return deg


def _tc_prescale(degp, x):
    n = x.shape[0]

    def body(degp_ref, x_ref, dinv_ref, xs_ref):
        d = degp_ref[0, :, 0:1] + degp_ref[1, :, 0:1] + 1.0
        dinv = lax.rsqrt(d)
        dinv_ref[...] = dinv
        xs_ref[...] = x_ref[...] * dinv

    return pl.pallas_call(
        body,
        grid=(n // RB,),
        in_specs=[
            pl.BlockSpec((NC, RB, 128), lambda i: (0, i, 0)),
            pl.BlockSpec((RB, 128), lambda i: (i, 0)),
        ],
        out_specs=[
            pl.BlockSpec((RB, 1), lambda i: (i, 0)),
            pl.BlockSpec((RB, 128), lambda i: (i, 0)),
        ],
        out_shape=[
            jax.ShapeDtypeStruct((n, 1), jnp.float32),
            jax.ShapeDtypeStruct((n, 128), jnp.float32),
        ],
    )(degp, x)


def _tc_layer_a(partials, base, dinv, w, b):
    n, k = base.shape
    hout = w.shape[1]

    def body(p_ref, base_ref, dinv_ref, w_ref, b_ref, pre_ref, stats_ref):
        i = pl.program_id(0)
        s = (p_ref[0] + p_ref[1] + base_ref[...]) * dinv_ref[...]
        pre = jnp.dot(s, w_ref[...], preferred_element_type=jnp.float32)
        pre = pre + b_ref[...]
        pre_ref[...] = pre
        su = jnp.sum(pre, axis=0, keepdims=True)
        sq = jnp.sum(pre * pre, axis=0, keepdims=True)
        upd = jnp.concatenate(
            [su, sq, jnp.zeros((6, hout), jnp.float32)], axis=0)

        @pl.when(i == 0)
        def _():
            stats_ref[...] = upd

        @pl.when(i > 0)
        def _():
            stats_ref[...] += upd

    return pl.pallas_call(
        body,
        grid=(n // RB,),
        in_specs=[
            pl.BlockSpec((NC, RB, 128), lambda i: (0, i, 0)),
            pl.BlockSpec((RB, 128), lambda i: (i, 0)),
            pl.BlockSpec((RB, 1), lambda i: (i, 0)),
            pl.BlockSpec((k, hout), lambda i: (0, 0)),
            pl.BlockSpec((1, hout), lambda i: (0, 0)),
        ],
        out_specs=[
            pl.BlockSpec((RB, hout), lambda i: (i, 0)),
            pl.BlockSpec((8, hout), lambda i: (0, 0)),
        ],
        out_shape=[
            jax.ShapeDtypeStruct((n, hout), jnp.float32),
            jax.ShapeDtypeStruct((8, hout), jnp.float32),
        ],
    )(partials, base, dinv, w, b.reshape(1, hout))


def _tc_layer_b(pre, stats, g, be, dinv, res=None):
    n, hout = pre.shape
    inv_n = 1.0 / n

    def bn_relu(pre_blk, stats_blk, g_blk, be_blk):
        mean = stats_blk[0:1, :] * inv_n
        var = stats_blk[1:2, :] * inv_n - mean * mean
        yn = (pre_blk - mean) * lax.rsqrt(var + 1e-5) * g_blk + be_blk
        return jnp.maximum(yn, 0.0)

    if res is None:
        def body(pre_ref, stats_ref, g_ref, be_ref, dinv_ref, y_ref, ys_ref):
            y = bn_relu(pre_ref[...], stats_ref[...], g_ref[...], be_ref[...])
            y_ref[...] = y
            ys_ref[...] = y * dinv_ref[...]
        extra_in = []
    else:
        def body(pre_ref, stats_ref, g_ref, be_ref, dinv_ref, res_ref,
                 y_ref, ys_ref):
            y = bn_relu(pre_ref[...], stats_ref[...], g_ref[...], be_ref[...])
            y = y + res_ref[...]
            y_ref[...] = y
            ys_ref[...] = y * dinv_ref[...]
        extra_in = [pl.BlockSpec((RB, hout), lambda i: (i, 0))]

    args = [pre, stats, g.reshape(1, hout), be.reshape(1, hout), dinv]
    if res is not None:
        args.append(res)
    return pl.pallas_call(
        body,
        grid=(n // RB,),
        in_specs=[
            pl.BlockSpec((RB, hout), lambda i: (i, 0)),
            pl.BlockSpec((8, hout), lambda i: (0, 0)),
            pl.BlockSpec((1, hout), lambda i: (0, 0)),
            pl.BlockSpec((1, hout), lambda i: (0, 0)),
            pl.BlockSpec((RB, 1), lambda i: (i, 0)),
        ] + extra_in,
        out_specs=[
            pl.BlockSpec((RB, hout), lambda i: (i, 0)),
            pl.BlockSpec((RB, hout), lambda i: (i, 0)),
        ],
        out_shape=[
            jax.ShapeDtypeStruct((n, hout), jnp.float32),
            jax.ShapeDtypeStruct((n, hout), jnp.float32),
        ],
    )(*args)


def _tc_heads(partials, base, dinv, wm, bm, ws, bs, eps):
    n = base.shape[0]
    lat = wm.shape[1]

    def body(p_ref, base_ref, dinv_ref, wm_ref, bm_ref, ws_ref, bs_ref,
             eps_ref, qz_ref, qm_ref, qs_ref):
        p = (p_ref[0] + p_ref[1] + base_ref[...]) * dinv_ref[...]
        qm = jnp.dot(p, wm_ref[...], preferred_element_type=jnp.float32)
        qm = qm + bm_ref[...]
        qs = jnp.dot(p, ws_ref[...], preferred_element_type=jnp.float32)
        qs = qs + bs_ref[...]
        std = jnp.logaddexp(qs, 0.0) + 1e-6
        qm_ref[...] = qm
        qs_ref[...] = qs
        qz_ref[...] = qm + std * eps_ref[...]

    return pl.pallas_call(
        body,
        grid=(n // RB,),
        in_specs=[
            pl.BlockSpec((NC, RB, 128), lambda i: (0, i, 0)),
            pl.BlockSpec((RB, 128), lambda i: (i, 0)),
            pl.BlockSpec((RB, 1), lambda i: (i, 0)),
            pl.BlockSpec((128, lat), lambda i: (0, 0)),
            pl.BlockSpec((1, lat), lambda i: (0, 0)),
            pl.BlockSpec((128, lat), lambda i: (0, 0)),
            pl.BlockSpec((1, lat), lambda i: (0, 0)),
            pl.BlockSpec((RB, lat), lambda i: (i, 0)),
        ],
        out_specs=[
            pl.BlockSpec((RB, lat), lambda i: (i, 0)),
            pl.BlockSpec((RB, lat), lambda i: (i, 0)),
            pl.BlockSpec((RB, lat), lambda i: (i, 0)),
        ],
        out_shape=[
            jax.ShapeDtypeStruct((n, lat), jnp.float32),
            jax.ShapeDtypeStruct((n, lat), jnp.float32),
            jax.ShapeDtypeStruct((n, lat), jnp.float32),
        ],
    )(partials, base, dinv, wm, bm.reshape(1, lat), ws, bs.reshape(1, lat),
      eps)


def kernel(x, edge_index, W1, b1, g1, be1, W2, b2, g2, be2, Wm, bm, Ws, bs):
    n = x.shape[0]
    e = edge_index.shape[1]
    lat = Wm.shape[1]

    n_acc = -(-(n + 1) // (NS * 8)) * (NS * 8)
    unit = CHUNK * NB
    epw = -(-e // (NW * unit)) * unit
    nch = epw // CHUNK
    pad = NW * epw - e

    src = edge_index[0].astype(jnp.int32)
    dst = edge_index[1].astype(jnp.int32)
    pad_src = jnp.arange(pad, dtype=jnp.int32) * 79 % n
    src3 = jnp.concatenate([src, pad_src])
    src3 = src3.reshape(NW, nch, CHUNK)
    pad_dst = n + jnp.arange(pad, dtype=jnp.int32) % (n_acc - n)
    dst3 = jnp.concatenate([dst, pad_dst])
    dst3 = dst3.reshape(NW, nch, CHUNK)

    zeros_w = jnp.zeros((n_acc, 128), jnp.float32)
    ones_d = jnp.ones((CHUNK, 128), jnp.float32)

    deg_fn = _make_deg(n_acc, nch)
    spmm_fn = _make_spmm(n_acc, nch, 128)

    degp = deg_fn(dst3, ones_d, zeros_w)
    dinv, xs = _tc_prescale(degp, x)

    s1 = spmm_fn(xs, src3, dst3, zeros_w)
    pre1, stats1 = _tc_layer_a(s1, xs, dinv, W1, b1)
    res, res_s = _tc_layer_b(pre1, stats1, g1, be1, dinv)

    s2 = spmm_fn(res_s, src3, dst3, zeros_w)
    pre2, stats2 = _tc_layer_a(s2, res_s, dinv, W2, b2)
    _, h_s = _tc_layer_b(pre2, stats2, g2, be2, dinv, res=res)

    s3 = spmm_fn(h_s, src3, dst3, zeros_w)
    eps = jax.random.normal(jax.random.key(1), (n, lat), jnp.float32)
    q_z, q_m, q_s = _tc_heads(s3, h_s, dinv, Wm, bm, Ws, bs, eps)
    return (q_z, q_m, q_s)


# speedup vs baseline: 1.0036x; 1.0036x over previous
import functools

import jax
import jax.numpy as jnp
from jax import lax
from jax.experimental import pallas as pl
from jax.experimental.pallas import tpu as pltpu
from jax.experimental.pallas import tpu_sc as plsc

NC = 2
NS = 16
NW = NC * NS
CHUNK = 128
NB = 4
RB = 2000


def _sc_mesh():
    return plsc.VectorSubcoreMesh(core_axis_name="c", subcore_axis_name="s")


def _make_spmm(n_acc, nch, width, nb=2):
    rpt = n_acc // NS
    if nb == 1:
        nwin = 1
    elif nb == 3 and nch % 3 == 0:
        nwin = 3
    else:
        nwin = 2 if nch % 2 == 0 else 1
    win = nch // nwin
    assert win % nb == 0

    @functools.partial(
        pl.kernel,
        mesh=_sc_mesh(),
        out_type=jax.ShapeDtypeStruct((NC, n_acc, width), jnp.float32),
        scratch_types=(
            [pltpu.VMEM((win, CHUNK), jnp.int32),
             pltpu.VMEM((win, CHUNK), jnp.int32)]
            + [pltpu.VMEM((CHUNK, width), jnp.float32)] * nb
            + [pltpu.VMEM_SHARED((n_acc, width), jnp.float32)]
            + [pltpu.SemaphoreType.DMA] * (2 * nb)
        ),
    )
    def spmm(table, src3, dst3, zeros, out, srcv, dstv, *rest):
        rows = rest[:nb]
        acc = rest[nb]
        gsem = rest[nb + 1:nb + 1 + nb]
        ssem = rest[nb + 1 + nb:]
        cid = lax.axis_index("c")
        sid = lax.axis_index("s")
        wid = sid * NC + cid
        sl = pl.ds(sid * rpt, rpt)
        pltpu.sync_copy(zeros.at[sl], acc.at[sl])
        plsc.subcore_barrier()

        for w in range(nwin):
            if nwin == 1:
                pltpu.sync_copy(src3.at[wid], srcv)
                pltpu.sync_copy(dst3.at[wid], dstv)
            else:
                pltpu.sync_copy(src3.at[wid, pl.ds(w * win, win)], srcv)
                pltpu.sync_copy(dst3.at[wid, pl.ds(w * win, win)], dstv)

            if nb == 1:
                def body(j, carry):
                    pltpu.async_copy(
                        table.at[srcv.at[j]], rows[0], gsem[0]).wait()
                    pltpu.sync_copy(rows[0], acc.at[dstv.at[j]], add=True)
                    return carry

                lax.fori_loop(0, win, body, 0)
            else:
                for b in range(nb):
                    pltpu.async_copy(table.at[srcv.at[b]], rows[b], gsem[b])

                def group(gi, carry):
                    for b in range(nb):
                        j = gi * nb + b
                        pltpu.make_async_copy(
                            table.at[srcv.at[j]], rows[b], gsem[b]).wait()
                        pltpu.async_copy(
                            rows[b], acc.at[dstv.at[j]], ssem[b], add=True)
                    for b in range(nb):
                        j = gi * nb + b
                        pltpu.make_async_copy(
                            rows[b], acc.at[dstv.at[j]], ssem[b]).wait()
                        jn = jnp.minimum(j + nb, win - 1)
                        pltpu.async_copy(
                            table.at[srcv.at[jn]], rows[b], gsem[b])
                    return carry

                lax.fori_loop(0, win // nb, group, 0)
                for b in range(nb):
                    pltpu.make_async_copy(
                        table.at[srcv.at[0]], rows[b], gsem[b]).wait()

        plsc.subcore_barrier()
        pltpu.sync_copy(acc.at[sl], out.at[cid, sl])

    return spmm


def _make_deg(n_acc, nch):
    rpt = n_acc // NS

    @functools.partial(
        pl.kernel,
        mesh=_sc_mesh(),
        out_type=jax.ShapeDtypeStruct((NC, n_acc, 128), jnp.float32),
        scratch_types=[
            pltpu.VMEM((nch, CHUNK), jnp.int32),
            pltpu.VMEM((CHUNK, 128), jnp.float32),
            pltpu.VMEM_SHARED((n_acc, 128), jnp.float32),
            pltpu.SemaphoreType.DMA,
        ],
    )
    def deg(dst3, ones, zeros, out, dstv, onesv, acc, sem):
        cid = lax.axis_index("c")
        sid = lax.axis_index("s")
        wid = sid * NC + cid
        sl = pl.ds(sid * rpt, rpt)
        pltpu.sync_copy(zeros.at[sl], acc.at[sl])
        pltpu.sync_copy(dst3.at[wid], dstv)
        pltpu.sync_copy(ones, onesv)
        plsc.subcore_barrier()

        k = next(d for d in (10, 9, 8, 7, 6, 5, 4, 3, 2, 1) if nch % d == 0)
        def group(gi, carry):
            for b in range(k):
                pltpu.async_copy(
                    onesv, acc.at[dstv.at[gi * k + b]], sem, add=True)
            for b in range(k):
                pltpu.make_async_copy(
                    onesv, acc.at[dstv.at[0]], sem).wait()
            return carry

        lax.fori_loop(0, nch // k, group, 0)
        plsc.subcore_barrier()
        pltpu.sync_copy(acc.at[sl], out.at[cid, sl])

    return deg


def _tc_prescale(degp, x):
    n = x.shape[0]

    def body(degp_ref, x_ref, dinv_ref, xs_ref):
        d = degp_ref[0, :, 0:1] + degp_ref[1, :, 0:1] + 1.0
        dinv = lax.rsqrt(d)
        dinv_ref[...] = dinv
        xs_ref[...] = x_ref[...] * dinv

    return pl.pallas_call(
        body,
        grid=(n // RB,),
        in_specs=[
            pl.BlockSpec((NC, RB, 128), lambda i: (0, i, 0)),
            pl.BlockSpec((RB, 128), lambda i: (i, 0)),
        ],
        out_specs=[
            pl.BlockSpec((RB, 1), lambda i: (i, 0)),
            pl.BlockSpec((RB, 128), lambda i: (i, 0)),
        ],
        out_shape=[
            jax.ShapeDtypeStruct((n, 1), jnp.float32),
            jax.ShapeDtypeStruct((n, 128), jnp.float32),
        ],
    )(degp, x)


def _tc_layer(partials, base, dinv, w, b, g, be, res=None):
    n, kdim = base.shape
    hout = w.shape[1]
    nblk = n // RB
    inv_n = 1.0 / n

    def body(p_ref, base_ref, dinv_ref, w_ref, b_ref, g_ref, be_ref,
             *rest):
        if res is None:
            y_ref, ys_ref, pre_scr, stats_scr = rest
            res_ref = None
        else:
            res_ref, y_ref, ys_ref, pre_scr, stats_scr = rest
        ph = pl.program_id(0)
        i = pl.program_id(1)

        @pl.when(ph == 0)
        def _():
            s = (p_ref[0] + p_ref[1] + base_ref[...]) * dinv_ref[...]
            pre = jnp.dot(s, w_ref[...], preferred_element_type=jnp.float32)
            pre = pre + b_ref[...]
            pre_scr[pl.ds(i * RB, RB), :] = pre
            su = jnp.sum(pre, axis=0, keepdims=True)
            sq = jnp.sum(pre * pre, axis=0, keepdims=True)
            upd = jnp.concatenate([su, sq], axis=0)

            @pl.when(i == 0)
            def _():
                stats_scr[...] = upd

            @pl.when(i > 0)
            def _():
                stats_scr[...] += upd

        @pl.when(ph == 1)
        def _():
            pre = pre_scr[pl.ds(i * RB, RB), :]
            mean = stats_scr[0:1, :] * inv_n
            var = stats_scr[1:2, :] * inv_n - mean * mean
            yn = (pre - mean) * lax.rsqrt(var + 1e-5) * g_ref[...]
            y = jnp.maximum(yn + be_ref[...], 0.0)
            if res_ref is not None:
                y = y + res_ref[...]
            y_ref[...] = y
            ys_ref[...] = y * dinv_ref[...]

    last = nblk - 1
    in_specs = [
        pl.BlockSpec((NC, RB, 128),
                     lambda p, i: (0, jnp.where(p == 0, i, last), 0)),
        pl.BlockSpec((RB, 128),
                     lambda p, i: (jnp.where(p == 0, i, last), 0)),
        pl.BlockSpec((RB, 1), lambda p, i: (i, 0)),
        pl.BlockSpec((kdim, hout), lambda p, i: (0, 0)),
        pl.BlockSpec((1, hout), lambda p, i: (0, 0)),
        pl.BlockSpec((1, hout), lambda p, i: (0, 0)),
        pl.BlockSpec((1, hout), lambda p, i: (0, 0)),
    ]
    args = [partials, base, dinv, w, b.reshape(1, hout),
            g.reshape(1, hout), be.reshape(1, hout)]
    if res is not None:
        in_specs.append(
            pl.BlockSpec((RB, hout),
                         lambda p, i: (jnp.where(p == 1, i, 0), 0)))
        args.append(res)
    return pl.pallas_call(
        body,
        grid=(2, nblk),
        in_specs=in_specs,
        out_specs=[
            pl.BlockSpec((RB, hout), lambda p, i: (i, 0)),
            pl.BlockSpec((RB, hout), lambda p, i: (i, 0)),
        ],
        out_shape=[
            jax.ShapeDtypeStruct((n, hout), jnp.float32),
            jax.ShapeDtypeStruct((n, hout), jnp.float32),
        ],
        scratch_shapes=[
            pltpu.VMEM((n, hout), jnp.float32),
            pltpu.VMEM((2, hout), jnp.float32),
        ],
    )(*args)


def _tc_layer_a(partials, base, dinv, w, b):
    n, k = base.shape
    hout = w.shape[1]

    def body(p_ref, base_ref, dinv_ref, w_ref, b_ref, pre_ref, stats_ref):
        i = pl.program_id(0)
        s = (p_ref[0] + p_ref[1] + base_ref[...]) * dinv_ref[...]
        pre = jnp.dot(s, w_ref[...], preferred_element_type=jnp.float32)
        pre = pre + b_ref[...]
        pre_ref[...] = pre
        su = jnp.sum(pre, axis=0, keepdims=True)
        sq = jnp.sum(pre * pre, axis=0, keepdims=True)
        upd = jnp.concatenate(
            [su, sq, jnp.zeros((6, hout), jnp.float32)], axis=0)

        @pl.when(i == 0)
        def _():
            stats_ref[...] = upd

        @pl.when(i > 0)
        def _():
            stats_ref[...] += upd

    return pl.pallas_call(
        body,
        grid=(n // RB,),
        in_specs=[
            pl.BlockSpec((NC, RB, 128), lambda i: (0, i, 0)),
            pl.BlockSpec((RB, 128), lambda i: (i, 0)),
            pl.BlockSpec((RB, 1), lambda i: (i, 0)),
            pl.BlockSpec((k, hout), lambda i: (0, 0)),
            pl.BlockSpec((1, hout), lambda i: (0, 0)),
        ],
        out_specs=[
            pl.BlockSpec((RB, hout), lambda i: (i, 0)),
            pl.BlockSpec((8, hout), lambda i: (0, 0)),
        ],
        out_shape=[
            jax.ShapeDtypeStruct((n, hout), jnp.float32),
            jax.ShapeDtypeStruct((8, hout), jnp.float32),
        ],
    )(partials, base, dinv, w, b.reshape(1, hout))


def _tc_layer_b(pre, stats, g, be, dinv, res=None):
    n, hout = pre.shape
    inv_n = 1.0 / n

    def bn_relu(pre_blk, stats_blk, g_blk, be_blk):
        mean = stats_blk[0:1, :] * inv_n
        var = stats_blk[1:2, :] * inv_n - mean * mean
        yn = (pre_blk - mean) * lax.rsqrt(var + 1e-5) * g_blk + be_blk
        return jnp.maximum(yn, 0.0)

    if res is None:
        def body(pre_ref, stats_ref, g_ref, be_ref, dinv_ref, y_ref, ys_ref):
            y = bn_relu(pre_ref[...], stats_ref[...], g_ref[...], be_ref[...])
            y_ref[...] = y
            ys_ref[...] = y * dinv_ref[...]
        extra_in = []
    else:
        def body(pre_ref, stats_ref, g_ref, be_ref, dinv_ref, res_ref,
                 y_ref, ys_ref):
            y = bn_relu(pre_ref[...], stats_ref[...], g_ref[...], be_ref[...])
            y = y + res_ref[...]
            y_ref[...] = y
            ys_ref[...] = y * dinv_ref[...]
        extra_in = [pl.BlockSpec((RB, hout), lambda i: (i, 0))]

    args = [pre, stats, g.reshape(1, hout), be.reshape(1, hout), dinv]
    if res is not None:
        args.append(res)
    return pl.pallas_call(
        body,
        grid=(n // RB,),
        in_specs=[
            pl.BlockSpec((RB, hout), lambda i: (i, 0)),
            pl.BlockSpec((8, hout), lambda i: (0, 0)),
            pl.BlockSpec((1, hout), lambda i: (0, 0)),
            pl.BlockSpec((1, hout), lambda i: (0, 0)),
            pl.BlockSpec((RB, 1), lambda i: (i, 0)),
        ] + extra_in,
        out_specs=[
            pl.BlockSpec((RB, hout), lambda i: (i, 0)),
            pl.BlockSpec((RB, hout), lambda i: (i, 0)),
        ],
        out_shape=[
            jax.ShapeDtypeStruct((n, hout), jnp.float32),
            jax.ShapeDtypeStruct((n, hout), jnp.float32),
        ],
    )(*args)


def _tc_heads(partials, base, dinv, wm, bm, ws, bs, eps):
    n = base.shape[0]
    lat = wm.shape[1]

    def body(p_ref, base_ref, dinv_ref, wm_ref, bm_ref, ws_ref, bs_ref,
             eps_ref, qz_ref, qm_ref, qs_ref):
        p = (p_ref[0] + p_ref[1] + base_ref[...]) * dinv_ref[...]
        qm = jnp.dot(p, wm_ref[...], preferred_element_type=jnp.float32)
        qm = qm + bm_ref[...]
        qs = jnp.dot(p, ws_ref[...], preferred_element_type=jnp.float32)
        qs = qs + bs_ref[...]
        std = jnp.logaddexp(qs, 0.0) + 1e-6
        qm_ref[...] = qm
        qs_ref[...] = qs
        qz_ref[...] = qm + std * eps_ref[...]

    return pl.pallas_call(
        body,
        grid=(n // RB,),
        in_specs=[
            pl.BlockSpec((NC, RB, 128), lambda i: (0, i, 0)),
            pl.BlockSpec((RB, 128), lambda i: (i, 0)),
            pl.BlockSpec((RB, 1), lambda i: (i, 0)),
            pl.BlockSpec((128, lat), lambda i: (0, 0)),
            pl.BlockSpec((1, lat), lambda i: (0, 0)),
            pl.BlockSpec((128, lat), lambda i: (0, 0)),
            pl.BlockSpec((1, lat), lambda i: (0, 0)),
            pl.BlockSpec((RB, lat), lambda i: (i, 0)),
        ],
        out_specs=[
            pl.BlockSpec((RB, lat), lambda i: (i, 0)),
            pl.BlockSpec((RB, lat), lambda i: (i, 0)),
            pl.BlockSpec((RB, lat), lambda i: (i, 0)),
        ],
        out_shape=[
            jax.ShapeDtypeStruct((n, lat), jnp.float32),
            jax.ShapeDtypeStruct((n, lat), jnp.float32),
            jax.ShapeDtypeStruct((n, lat), jnp.float32),
        ],
    )(partials, base, dinv, wm, bm.reshape(1, lat), ws, bs.reshape(1, lat),
      eps)


def kernel(x, edge_index, W1, b1, g1, be1, W2, b2, g2, be2, Wm, bm, Ws, bs):
    n = x.shape[0]
    e = edge_index.shape[1]
    lat = Wm.shape[1]

    n_acc = -(-(n + 1) // (NS * 8)) * (NS * 8)
    unit = CHUNK * NB
    epw = -(-e // (NW * unit)) * unit
    nch = epw // CHUNK
    pad = NW * epw - e

    src = edge_index[0].astype(jnp.int32)
    dst = edge_index[1].astype(jnp.int32)
    pad_src = jnp.arange(pad, dtype=jnp.int32) * 79 % n
    src3 = jnp.concatenate([src, pad_src])
    src3 = src3.reshape(NW, nch, CHUNK)
    pad_dst = n + jnp.arange(pad, dtype=jnp.int32) % (n_acc - n)
    dst3 = jnp.concatenate([dst, pad_dst])
    dst3 = dst3.reshape(NW, nch, CHUNK)

    zeros_w = jnp.zeros((n_acc, 128), jnp.float32)
    ones_d = jnp.ones((CHUNK, 128), jnp.float32)

    deg_fn = _make_deg(n_acc, nch)
    spmm_fn = _make_spmm(n_acc, nch, 128)

    degp = deg_fn(dst3, ones_d, zeros_w)
    dinv, xs = _tc_prescale(degp, x)

    s1 = spmm_fn(xs, src3, dst3, zeros_w)
    res, res_s = _tc_layer(s1, xs, dinv, W1, b1, g1, be1)

    s2 = spmm_fn(res_s, src3, dst3, zeros_w)
    _, h_s = _tc_layer(s2, res_s, dinv, W2, b2, g2, be2, res=res)

    s3 = spmm_fn(h_s, src3, dst3, zeros_w)
    eps = jax.random.normal(jax.random.key(1), (n, lat), jnp.float32)
    q_z, q_m, q_s = _tc_heads(s3, h_s, dinv, Wm, bm, Ws, bs, eps)
    return (q_z, q_m, q_s)


# speedup vs baseline: 1.0051x; 1.0014x over previous
import functools

import jax
import jax.numpy as jnp
from jax import lax
from jax.experimental import pallas as pl
from jax.experimental.pallas import tpu as pltpu
from jax.experimental.pallas import tpu_sc as plsc

NC = 2
NS = 16
NW = NC * NS
CHUNK = 128
NB = 4
RB = 2000


def _sc_mesh():
    return plsc.VectorSubcoreMesh(core_axis_name="c", subcore_axis_name="s")


def _make_spmm(n_acc, nch, width, nb=2):
    rpt = n_acc // NS
    if nb == 1:
        nwin = 1
    elif nb == 3 and nch % 3 == 0:
        nwin = 3
    else:
        nwin = 2 if nch % 2 == 0 else 1
    win = nch // nwin
    assert win % nb == 0

    @functools.partial(
        pl.kernel,
        mesh=_sc_mesh(),
        out_type=jax.ShapeDtypeStruct((NC, n_acc, width), jnp.float32),
        scratch_types=(
            [pltpu.VMEM((win, CHUNK), jnp.int32),
             pltpu.VMEM((win, CHUNK), jnp.int32)]
            + [pltpu.VMEM((CHUNK, width), jnp.float32)] * nb
            + [pltpu.VMEM_SHARED((n_acc, width), jnp.float32)]
            + [pltpu.SemaphoreType.DMA] * (2 * nb)
        ),
    )
    def spmm(table, src3, dst3, zeros, out, srcv, dstv, *rest):
        rows = rest[:nb]
        acc = rest[nb]
        gsem = rest[nb + 1:nb + 1 + nb]
        ssem = rest[nb + 1 + nb:]
        cid = lax.axis_index("c")
        sid = lax.axis_index("s")
        wid = sid * NC + cid
        sl = pl.ds(sid * rpt, rpt)
        pltpu.sync_copy(zeros.at[sl], acc.at[sl])
        plsc.subcore_barrier()

        for w in range(nwin):
            if nwin == 1:
                pltpu.sync_copy(src3.at[wid], srcv)
                pltpu.sync_copy(dst3.at[wid], dstv)
            else:
                pltpu.sync_copy(src3.at[wid, pl.ds(w * win, win)], srcv)
                pltpu.sync_copy(dst3.at[wid, pl.ds(w * win, win)], dstv)

            if nb == 1:
                def body(j, carry):
                    pltpu.async_copy(
                        table.at[srcv.at[j]], rows[0], gsem[0]).wait()
                    pltpu.sync_copy(rows[0], acc.at[dstv.at[j]], add=True)
                    return carry

                lax.fori_loop(0, win, body, 0)
            else:
                for b in range(nb):
                    pltpu.async_copy(table.at[srcv.at[b]], rows[b], gsem[b])

                def group(gi, carry):
                    for b in range(nb):
                        j = gi * nb + b
                        pltpu.make_async_copy(
                            table.at[srcv.at[j]], rows[b], gsem[b]).wait()
                        pltpu.async_copy(
                            rows[b], acc.at[dstv.at[j]], ssem[b], add=True)
                    for b in range(nb):
                        j = gi * nb + b
                        pltpu.make_async_copy(
                            rows[b], acc.at[dstv.at[j]], ssem[b]).wait()
                        jn = jnp.minimum(j + nb, win - 1)
                        pltpu.async_copy(
                            table.at[srcv.at[jn]], rows[b], gsem[b])
                    return carry

                lax.fori_loop(0, win // nb, group, 0)
                for b in range(nb):
                    pltpu.make_async_copy(
                        table.at[srcv.at[0]], rows[b], gsem[b]).wait()

        plsc.subcore_barrier()
        pltpu.sync_copy(acc.at[sl], out.at[cid, sl])

    return spmm


def _make_deg(n_acc, nch):
    rpt = n_acc // NS

    @functools.partial(
        pl.kernel,
        mesh=_sc_mesh(),
        out_type=jax.ShapeDtypeStruct((NC, n_acc, 128), jnp.float32),
        scratch_types=[
            pltpu.VMEM((nch, CHUNK), jnp.int32),
            pltpu.VMEM((CHUNK, 128), jnp.float32),
            pltpu.VMEM_SHARED((n_acc, 128), jnp.float32),
            pltpu.SemaphoreType.DMA,
        ],
    )
    def deg(dst3, ones, zeros, out, dstv, onesv, acc, sem):
        cid = lax.axis_index("c")
        sid = lax.axis_index("s")
        wid = sid * NC + cid
        sl = pl.ds(sid * rpt, rpt)
        pltpu.sync_copy(zeros.at[sl], acc.at[sl])
        pltpu.sync_copy(dst3.at[wid], dstv)
        pltpu.sync_copy(ones, onesv)
        plsc.subcore_barrier()

        k = next(d for d in (10, 9, 8, 7, 6, 5, 4, 3, 2, 1) if nch % d == 0)
        def group(gi, carry):
            for b in range(k):
                pltpu.async_copy(
                    onesv, acc.at[dstv.at[gi * k + b]], sem, add=True)
            for b in range(k):
                pltpu.make_async_copy(
                    onesv, acc.at[dstv.at[0]], sem).wait()
            return carry

        lax.fori_loop(0, nch // k, group, 0)
        plsc.subcore_barrier()
        pltpu.sync_copy(acc.at[sl], out.at[cid, sl])

    return deg


def _tc_prescale(degp, x):
    n = x.shape[0]

    def body(degp_ref, x_ref, dinv_ref, xs_ref):
        d = degp_ref[0, :, 0:1] + degp_ref[1, :, 0:1] + 1.0
        dinv = lax.rsqrt(d)
        dinv_ref[...] = dinv
        xs_ref[...] = x_ref[...] * dinv

    return pl.pallas_call(
        body,
        grid=(n // RB,),
        in_specs=[
            pl.BlockSpec((NC, RB, 128), lambda i: (0, i, 0)),
            pl.BlockSpec((RB, 128), lambda i: (i, 0)),
        ],
        out_specs=[
            pl.BlockSpec((RB, 1), lambda i: (i, 0)),
            pl.BlockSpec((RB, 128), lambda i: (i, 0)),
        ],
        out_shape=[
            jax.ShapeDtypeStruct((n, 1), jnp.float32),
            jax.ShapeDtypeStruct((n, 128), jnp.float32),
        ],
    )(degp, x)


def _tc_layer(partials, base, dinv, w, b, g, be, res=None):
    n, kdim = base.shape
    hout = w.shape[1]
    nblk = n // RB
    inv_n = 1.0 / n

    def body(p_ref, base_ref, dinv_ref, w_ref, b_ref, g_ref, be_ref,
             *rest):
        if res is None:
            y_ref, ys_ref, pre_scr, stats_scr = rest
            res_ref = None
        else:
            res_ref, y_ref, ys_ref, pre_scr, stats_scr = rest
        ph = pl.program_id(0)
        i = pl.program_id(1)

        @pl.when(ph == 0)
        def _():
            s = (p_ref[0] + p_ref[1] + base_ref[...]) * dinv_ref[...]
            pre = jnp.dot(s, w_ref[...], preferred_element_type=jnp.float32)
            pre = pre + b_ref[...]
            pre_scr[pl.ds(i * RB, RB), :] = pre
            su = jnp.sum(pre, axis=0, keepdims=True)
            sq = jnp.sum(pre * pre, axis=0, keepdims=True)
            upd = jnp.concatenate([su, sq], axis=0)

            @pl.when(i == 0)
            def _():
                stats_scr[...] = upd

            @pl.when(i > 0)
            def _():
                stats_scr[...] += upd

        @pl.when(ph == 1)
        def _():
            pre = pre_scr[pl.ds(i * RB, RB), :]
            mean = stats_scr[0:1, :] * inv_n
            var = stats_scr[1:2, :] * inv_n - mean * mean
            yn = (pre - mean) * lax.rsqrt(var + 1e-5) * g_ref[...]
            y = jnp.maximum(yn + be_ref[...], 0.0)
            if res_ref is not None:
                y = y + res_ref[...]
            y_ref[...] = y
            ys_ref[...] = y * dinv_ref[...]

    last = nblk - 1
    in_specs = [
        pl.BlockSpec((NC, RB, 128),
                     lambda p, i: (0, jnp.where(p == 0, i, last), 0)),
        pl.BlockSpec((RB, 128),
                     lambda p, i: (jnp.where(p == 0, i, last), 0)),
        pl.BlockSpec((RB, 1), lambda p, i: (i, 0)),
        pl.BlockSpec((kdim, hout), lambda p, i: (0, 0)),
        pl.BlockSpec((1, hout), lambda p, i: (0, 0)),
        pl.BlockSpec((1, hout), lambda p, i: (0, 0)),
        pl.BlockSpec((1, hout), lambda p, i: (0, 0)),
    ]
    args = [partials, base, dinv, w, b.reshape(1, hout),
            g.reshape(1, hout), be.reshape(1, hout)]
    if res is not None:
        in_specs.append(
            pl.BlockSpec((RB, hout),
                         lambda p, i: (jnp.where(p == 1, i, 0), 0)))
        args.append(res)
    return pl.pallas_call(
        body,
        grid=(2, nblk),
        in_specs=in_specs,
        out_specs=[
            pl.BlockSpec((RB, hout), lambda p, i: (i, 0)),
            pl.BlockSpec((RB, hout), lambda p, i: (i, 0)),
        ],
        out_shape=[
            jax.ShapeDtypeStruct((n, hout), jnp.float32),
            jax.ShapeDtypeStruct((n, hout), jnp.float32),
        ],
        scratch_shapes=[
            pltpu.VMEM((n, hout), jnp.float32),
            pltpu.VMEM((2, hout), jnp.float32),
        ],
    )(*args)


def _tc_heads(partials, base, dinv, wm, bm, ws, bs, eps):
    n = base.shape[0]
    lat = wm.shape[1]

    def body(p_ref, base_ref, dinv_ref, wm_ref, bm_ref, ws_ref, bs_ref,
             eps_ref, qz_ref, qm_ref, qs_ref):
        p = (p_ref[0] + p_ref[1] + base_ref[...]) * dinv_ref[...]
        qm = jnp.dot(p, wm_ref[...], preferred_element_type=jnp.float32)
        qm = qm + bm_ref[...]
        qs = jnp.dot(p, ws_ref[...], preferred_element_type=jnp.float32)
        qs = qs + bs_ref[...]
        std = jnp.logaddexp(qs, 0.0) + 1e-6
        qm_ref[...] = qm
        qs_ref[...] = qs
        qz_ref[...] = qm + std * eps_ref[...]

    return pl.pallas_call(
        body,
        grid=(n // RB,),
        in_specs=[
            pl.BlockSpec((NC, RB, 128), lambda i: (0, i, 0)),
            pl.BlockSpec((RB, 128), lambda i: (i, 0)),
            pl.BlockSpec((RB, 1), lambda i: (i, 0)),
            pl.BlockSpec((128, lat), lambda i: (0, 0)),
            pl.BlockSpec((1, lat), lambda i: (0, 0)),
            pl.BlockSpec((128, lat), lambda i: (0, 0)),
            pl.BlockSpec((1, lat), lambda i: (0, 0)),
            pl.BlockSpec((RB, lat), lambda i: (i, 0)),
        ],
        out_specs=[
            pl.BlockSpec((RB, lat), lambda i: (i, 0)),
            pl.BlockSpec((RB, lat), lambda i: (i, 0)),
            pl.BlockSpec((RB, lat), lambda i: (i, 0)),
        ],
        out_shape=[
            jax.ShapeDtypeStruct((n, lat), jnp.float32),
            jax.ShapeDtypeStruct((n, lat), jnp.float32),
            jax.ShapeDtypeStruct((n, lat), jnp.float32),
        ],
    )(partials, base, dinv, wm, bm.reshape(1, lat), ws, bs.reshape(1, lat),
      eps)


def kernel(x, edge_index, W1, b1, g1, be1, W2, b2, g2, be2, Wm, bm, Ws, bs):
    n = x.shape[0]
    e = edge_index.shape[1]
    lat = Wm.shape[1]

    n_acc = -(-(n + 1) // (NS * 8)) * (NS * 8)
    unit = CHUNK * NB
    epw = -(-e // (NW * unit)) * unit
    nch = epw // CHUNK
    pad = NW * epw - e

    src = edge_index[0].astype(jnp.int32)
    dst = edge_index[1].astype(jnp.int32)
    pad_src = jnp.arange(pad, dtype=jnp.int32) * 79 % n
    src3 = jnp.concatenate([src, pad_src])
    src3 = src3.reshape(NW, nch, CHUNK)
    pad_dst = n + jnp.arange(pad, dtype=jnp.int32) % (n_acc - n)
    dst3 = jnp.concatenate([dst, pad_dst])
    dst3 = dst3.reshape(NW, nch, CHUNK)

    zeros_w = jnp.zeros((n_acc, 128), jnp.float32)
    ones_d = jnp.ones((CHUNK, 128), jnp.float32)

    deg_fn = _make_deg(n_acc, nch)
    spmm_fn = _make_spmm(n_acc, nch, 128)

    degp = deg_fn(dst3, ones_d, zeros_w)
    dinv, xs = _tc_prescale(degp, x)

    s1 = spmm_fn(xs, src3, dst3, zeros_w)
    res, res_s = _tc_layer(s1, xs, dinv, W1, b1, g1, be1)

    s2 = spmm_fn(res_s, src3, dst3, zeros_w)
    _, h_s = _tc_layer(s2, res_s, dinv, W2, b2, g2, be2, res=res)

    s3 = spmm_fn(h_s, src3, dst3, zeros_w)
    eps = jax.random.normal(jax.random.key(1), (n, lat), jnp.float32)
    q_z, q_m, q_s = _tc_heads(s3, h_s, dinv, Wm, bm, Ws, bs, eps)
    return (q_z, q_m, q_s)
